# baseline (speedup 1.0000x reference)
import functools
import jax
import jax.numpy as jnp
from jax import lax
from jax.experimental import pallas as pl
from jax.experimental.pallas import tpu as pltpu
from jax.experimental.pallas import tpu_sc as plsc

_C = 1000
_D1 = 26
_B = 16384
_NW = 32
_ROWS = _D1 * (_C // 8)
_RW = _ROWS // _NW
_REM = _ROWS - _RW * _NW
_QW = 32768
_NQ = 4


def _sc_onehot(xt_hbm, out_hbm, xrow, buf0, buf1, sem0, sem1):
    wid = lax.axis_index("s") * 2 + lax.axis_index("c")
    start = wid * _RW + jnp.minimum(wid, _REM)
    cnt = _RW + jnp.where(wid < _REM, 1, 0)
    j0 = start // 125
    kt0 = start - j0 * 125

    lanes = lax.iota(jnp.int32, 16)
    ones = jnp.ones((16,), jnp.float32)
    zeros = jnp.zeros((16,), jnp.float32)
    dump = lanes + _QW
    bufs = (buf0, buf1)
    sems = (sem0, sem1)

    pltpu.sync_copy(xt_hbm.at[pl.ds(j0 * _B, _B)], xrow.at[pl.ds(0, _B)])

    def _z(t, _):
        for u in range(8):
            o = (t * 8 + u) * 16
            buf0[pl.ds(o, 16)] = zeros
            buf1[pl.ds(o, 16)] = zeros
        return 0
    lax.fori_loop(0, (_QW + 128) // 128, _z, 0)

    def _scan(buf, kt, xb, val):
        def _s(g, _):
            for u in range(4):
                gg = g * 4 + u
                base = (gg >> 3) * 1024 + (gg & 7) * 16
                xs = xrow[pl.ds(xb + gg * 16, 16)]
                w = (lanes + base) + ((xs & 7) << 7)
                m = (xs >> 3) == kt
                w = jnp.where(m, w, dump)
                plsc.store_scatter(buf, [w], val)
            return 0
        lax.fori_loop(0, 4096 // 64, _s, 0)

    def _row(r, carry):
        j, kt, nst, okt0, oxb0, okt1, oxb1 = carry
        okts = [okt0, okt1]
        oxbs = [oxb0, oxb1]

        stg = jnp.logical_and(r > 0, kt == 0)

        @pl.when(stg)
        def _():
            pltpu.sync_copy(xt_hbm.at[pl.ds(j * _B, _B)],
                            xrow.at[pl.ds(_B, _B)])

        nst = jnp.where(stg, nst + 1, nst)
        jsel = nst - 1

        row_off = (j * 125 + kt) * (_NQ * _QW)
        for q in range(_NQ):
            c = r * _NQ + q
            b = q % 2
            buf, sem = bufs[b], sems[b]
            dst = out_hbm.at[pl.ds(row_off + q * _QW, _QW)]
            xb = jsel * _B + q * 4096

            @pl.when(c >= 2)
            def _():
                pltpu.make_async_copy(buf.at[pl.ds(0, _QW)], dst, sem).wait()
                _scan(buf, okts[b], oxbs[b], zeros)

            _scan(buf, kt, xb, ones)
            okts[b] = kt
            oxbs[b] = xb

            pltpu.async_copy(buf.at[pl.ds(0, _QW)], dst, sem)

        kt = kt + 1
        wrap = kt == 125
        j = jnp.where(wrap, j + 1, j)
        kt = jnp.where(wrap, 0, kt)
        return (j, kt, nst, okts[0], oxbs[0], okts[1], oxbs[1])

    z = jnp.int32(0)
    lax.fori_loop(0, cnt, _row, (j0, kt0, jnp.int32(1), z, z, z, z))

    dummy = out_hbm.at[pl.ds(0, _QW)]
    for b in range(2):
        pltpu.make_async_copy(bufs[b].at[pl.ds(0, _QW)], dummy, sems[b]).wait()


def kernel(x):
    b, c = x.shape
    xt = x.T.astype(jnp.int32).reshape(b * c)
    k = functools.partial(
        pl.kernel,
        mesh=plsc.VectorSubcoreMesh(core_axis_name="c", subcore_axis_name="s"),
        compiler_params=pltpu.CompilerParams(needs_layout_passes=False),
        out_type=jax.ShapeDtypeStruct((b * c * _C,), jnp.float32),
        scratch_types=[
            pltpu.VMEM((2 * _B,), jnp.int32),
            pltpu.VMEM((_QW + 128,), jnp.float32),
            pltpu.VMEM((_QW + 128,), jnp.float32),
            pltpu.SemaphoreType.DMA,
            pltpu.SemaphoreType.DMA,
        ],
    )(_sc_onehot)
    out = k(xt)
    out5 = out.reshape(_D1, _C // 8, _B // 128, 8, 128)
    return out5.transpose(2, 4, 0, 1, 3).reshape(_B, _D1, _C)


# speedup vs baseline: 1.3163x; 1.3163x over previous
import functools
import jax
import jax.numpy as jnp
from jax import lax
from jax.experimental import pallas as pl
from jax.experimental.pallas import tpu as pltpu
from jax.experimental.pallas import tpu_sc as plsc

_C = 1000
_D1 = 26
_B = 16384
_NW = 32
_ROWS = _D1 * (_C // 8)
_RW = _ROWS // _NW
_REM = _ROWS - _RW * _NW
_QW = 16384
_NQ = 8
_NS = 4


def _sc_onehot(xt_hbm, out_hbm, xrow, buf0, buf1, buf2, buf3,
               sem0, sem1, sem2, sem3):
    wid = lax.axis_index("s") * 2 + lax.axis_index("c")
    start = wid * _RW + jnp.minimum(wid, _REM)
    cnt = _RW + jnp.where(wid < _REM, 1, 0)
    j0 = start // 125
    kt0 = start - j0 * 125

    lanes = lax.iota(jnp.int32, 16)
    ones = jnp.ones((16,), jnp.float32)
    zeros = jnp.zeros((16,), jnp.float32)
    dump = lanes + _QW
    bufs = (buf0, buf1, buf2, buf3)
    sems = (sem0, sem1, sem2, sem3)

    pltpu.sync_copy(xt_hbm.at[pl.ds(j0 * _B, _B)], xrow.at[pl.ds(0, _B)])

    def _row(r, carry):
        j, kt, nst = carry

        stg = jnp.logical_and(r > 0, kt == 0)

        @pl.when(stg)
        def _():
            pltpu.sync_copy(xt_hbm.at[pl.ds(j * _B, _B)],
                            xrow.at[pl.ds(_B, _B)])

        nst = jnp.where(stg, nst + 1, nst)
        jsel = nst - 1

        row_off = (j * 125 + kt) * (_NQ * _QW)
        for q in range(_NQ):
            c = r * _NQ + q
            b = q % _NS
            buf, sem = bufs[b], sems[b]
            dst = out_hbm.at[pl.ds(row_off + q * _QW, _QW)]

            @pl.when(c >= _NS)
            def _():
                pltpu.make_async_copy(buf.at[pl.ds(0, _QW)], dst, sem).wait()

            def _z(t, _):
                for u in range(32):
                    buf[pl.ds((t * 32 + u) * 16, 16)] = zeros
                return 0
            lax.fori_loop(0, _QW // 512, _z, 0)

            xb = jsel * _B + q * 2048
            def _s(g, _):
                for u in range(4):
                    gg = g * 4 + u
                    base = (gg >> 3) * 1024 + (gg & 7) * 16
                    xs = xrow[pl.ds(xb + gg * 16, 16)]
                    w = (lanes + base) + ((xs & 7) << 7)
                    m = (xs >> 3) == kt
                    w = jnp.where(m, w, dump)
                    plsc.store_scatter(buf, [w], ones)
                return 0
            lax.fori_loop(0, 2048 // 64, _s, 0)

            pltpu.async_copy(buf.at[pl.ds(0, _QW)], dst, sem)

        kt = kt + 1
        wrap = kt == 125
        j = jnp.where(wrap, j + 1, j)
        kt = jnp.where(wrap, 0, kt)
        return (j, kt, nst)

    lax.fori_loop(0, cnt, _row, (j0, kt0, jnp.int32(1)))

    dummy = out_hbm.at[pl.ds(0, _QW)]
    for b in range(_NS):
        pltpu.make_async_copy(bufs[b].at[pl.ds(0, _QW)], dummy, sems[b]).wait()


def kernel(x):
    b, c = x.shape
    xt = x.T.astype(jnp.int32).reshape(b * c)
    k = functools.partial(
        pl.kernel,
        mesh=plsc.VectorSubcoreMesh(core_axis_name="c", subcore_axis_name="s"),
        compiler_params=pltpu.CompilerParams(needs_layout_passes=False),
        out_type=jax.ShapeDtypeStruct((b * c * _C,), jnp.float32),
        scratch_types=[
            pltpu.VMEM((2 * _B,), jnp.int32),
            pltpu.VMEM((_QW + 128,), jnp.float32),
            pltpu.VMEM((_QW + 128,), jnp.float32),
            pltpu.VMEM((_QW + 128,), jnp.float32),
            pltpu.VMEM((_QW + 128,), jnp.float32),
            pltpu.SemaphoreType.DMA,
            pltpu.SemaphoreType.DMA,
            pltpu.SemaphoreType.DMA,
            pltpu.SemaphoreType.DMA,
        ],
    )(_sc_onehot)
    out = k(xt)
    out5 = out.reshape(_D1, _C // 8, _B // 128, 8, 128)
    return out5.transpose(2, 4, 0, 1, 3).reshape(_B, _D1, _C)


# speedup vs baseline: 2.0235x; 1.5372x over previous
import functools
import jax
import jax.numpy as jnp
from jax import lax
from jax.experimental import pallas as pl
from jax.experimental.pallas import tpu as pltpu
from jax.experimental.pallas import tpu_sc as plsc

_C = 1000
_D1 = 26
_B = 16384
_NW = 32
_N = _D1 * _C * _B
_IW = _B // _NW
_OPW = _IW * _D1
_ZBLK = _N // 128


def _zero_block(o_ref):
    o_ref[...] = jnp.zeros(o_ref.shape, jnp.float32)


def _sc_ones(xt_hbm, out_ref, xall, wlist, onesv, sem):
    wid = lax.axis_index("s") * 2 + lax.axis_index("c")
    i0 = wid * _IW
    lanes = lax.iota(jnp.int32, 16)
    ones = jnp.ones((16,), jnp.float32)

    for j in range(_D1):
        pltpu.sync_copy(xt_hbm.at[pl.ds(j * _B + i0, _IW)],
                        xall.at[pl.ds(j * _IW, _IW)])

    for j in range(_D1):
        jbase = j * 125

        def _bj(g, _):
            idx = j * _IW + g * 16
            xs = xall[pl.ds(idx, 16)]
            i = i0 + g * 16 + lanes
            w = (((jbase + (xs >> 3)) << 17) + ((i >> 7) << 10)
                 + ((xs & 7) << 7) + (i & 127))
            wlist[pl.ds(idx, 16)] = w
            onesv[pl.ds(idx, 16)] = ones
            return 0

        lax.fori_loop(0, _IW // 16, _bj, 0)

    pltpu.async_copy(onesv, out_ref.at[wlist], sem).wait()


def kernel(x):
    b, c = x.shape
    xt = x.T.astype(jnp.int32).reshape(b * c)

    zeros_flat = pl.pallas_call(
        _zero_block,
        grid=(_N // _ZBLK,),
        out_specs=pl.BlockSpec((_ZBLK,), lambda i: (i,)),
        out_shape=jax.ShapeDtypeStruct((_N,), jnp.float32),
    )()

    scatter = functools.partial(
        pl.kernel,
        mesh=plsc.VectorSubcoreMesh(core_axis_name="c", subcore_axis_name="s"),
        compiler_params=pltpu.CompilerParams(needs_layout_passes=False),
        out_type=(),
        scratch_types=[
            pltpu.VMEM((_OPW,), jnp.int32),
            pltpu.VMEM((_OPW,), jnp.int32),
            pltpu.VMEM((_OPW,), jnp.float32),
            pltpu.SemaphoreType.DMA,
        ],
    )(_sc_ones)

    def run(xt_arr, zf):
        r = jax.new_ref(zf)
        scatter(xt_arr, r)
        return r[...]

    out = run(xt, zeros_flat)
    out5 = out.reshape(_D1, _C // 8, _B // 128, 8, 128)
    return out5.transpose(2, 4, 0, 1, 3).reshape(_B, _D1, _C)
